# Initial kernel scaffold; baseline (speedup 1.0000x reference)
import jax
import jax.numpy as jnp
from jax.experimental import pallas as pl


def kernel(pred, durations, events):
    raise NotImplementedError("write your pallas kernel here")


# speedup vs baseline: 3.4748x; 3.4748x over previous
import functools

import jax
import jax.numpy as jnp
from jax import lax
from jax.experimental import pallas as pl
from jax.experimental.pallas import tpu as pltpu
from jax.experimental.pallas import tpu_sc as plsc

N = 65536
NBUCKETS = 512
NC = 2
NS = 16
NW = NC * NS
PER_W = N // NW
LANES = 16
CHUNKS = PER_W // LANES
ACC_WORDS = 3 * LANES * NBUCKETS


def _sc_binning_kernel(pred_hbm, dur_hbm, ev_hbm, zeros_hbm, out_hbm,
                       pred_v, dur_v, ev_v, acc_v):
    wid = lax.axis_index("s") * NC + lax.axis_index("c")
    base = wid * PER_W
    pltpu.sync_copy(pred_hbm.at[pl.ds(base, PER_W)], pred_v)
    pltpu.sync_copy(dur_hbm.at[pl.ds(base, PER_W)], dur_v)
    pltpu.sync_copy(ev_hbm.at[pl.ds(base, PER_W)], ev_v)
    pltpu.sync_copy(zeros_hbm, acc_v)

    lane_off = lax.broadcasted_iota(jnp.int32, (LANES,), 0) * NBUCKETS

    def body(i, _):
        s = pl.ds(i * LANES, LANES)
        y = pred_v[s]
        d = dur_v[s]
        e = ev_v[s].astype(jnp.float32)
        expy = jnp.exp(jnp.clip(y, -20.0, 20.0))
        addr = lane_off + d
        plsc.addupdate_scatter(acc_v, [addr], expy)
        plsc.addupdate_scatter(acc_v, [addr + (LANES * NBUCKETS)], y * e)
        plsc.addupdate_scatter(acc_v, [addr + (2 * LANES * NBUCKETS)], e)
        return ()

    lax.fori_loop(0, CHUNKS, body, ())
    pltpu.sync_copy(acc_v, out_hbm.at[wid])


def _sc_binning(pred, durations, events, zeros):
    mesh = plsc.VectorSubcoreMesh(core_axis_name="c", subcore_axis_name="s")
    kern = functools.partial(
        pl.kernel,
        mesh=mesh,
        out_type=jax.ShapeDtypeStruct((NW, ACC_WORDS), jnp.float32),
        scratch_types=[
            pltpu.VMEM((PER_W,), jnp.float32),
            pltpu.VMEM((PER_W,), jnp.int32),
            pltpu.VMEM((PER_W,), jnp.int32),
            pltpu.VMEM((ACC_WORDS,), jnp.float32),
        ],
        compiler_params=pltpu.CompilerParams(needs_layout_passes=False),
    )(_sc_binning_kernel)
    return kern(pred, durations, events, zeros)


def _tc_finalize_kernel(x_ref, o_ref):
    x = x_ref[...]
    rows = x.shape[0]
    q = (lax.broadcasted_iota(jnp.int32, (rows, 1), 0) // LANES) % 3
    s_exp = jnp.sum(jnp.where(q == 0, x, 0.0), axis=0, keepdims=True)
    s_ye = jnp.sum(jnp.where(q == 1, x, 0.0), axis=0, keepdims=True)
    s_e = jnp.sum(jnp.where(q == 2, x, 0.0), axis=0, keepdims=True)
    jj = lax.broadcasted_iota(jnp.int32, (NBUCKETS, NBUCKETS), 0)
    ii = lax.broadcasted_iota(jnp.int32, (NBUCKETS, NBUCKETS), 1)
    tri = (jj >= ii).astype(jnp.float32)
    risk = jnp.dot(s_exp, tri, precision=lax.Precision.HIGHEST,
                   preferred_element_type=jnp.float32)
    ll = s_ye - s_e * jnp.log(jnp.maximum(risk, 1e-12))
    total_ll = jnp.sum(jnp.where(s_e > 0.0, ll, 0.0))
    n_events = jnp.maximum(jnp.sum(s_e), 1.0)
    o_ref[...] = jnp.broadcast_to(-total_ll / n_events, (1, 1))


def kernel(pred, durations, events):
    zeros = jnp.zeros((ACC_WORDS,), jnp.float32)
    parts = _sc_binning(pred.reshape(-1).astype(jnp.float32),
                        durations.reshape(-1), events.reshape(-1), zeros)
    x = parts.reshape(NW * 3 * LANES, NBUCKETS)
    out = pl.pallas_call(
        _tc_finalize_kernel,
        out_shape=jax.ShapeDtypeStruct((1, 1), jnp.float32),
    )(x)
    return out.reshape(1)


# speedup vs baseline: 4.1663x; 1.1990x over previous
import functools

import jax
import jax.numpy as jnp
from jax import lax
from jax.experimental import pallas as pl
from jax.experimental.pallas import tpu as pltpu
from jax.experimental.pallas import tpu_sc as plsc

N = 65536
NBUCKETS = 512
NC = 2
NS = 16
NW = NC * NS
PER_W = N // NW
LANES = 16
CHUNKS = PER_W // LANES
LB = LANES * NBUCKETS
ACC_WORDS = 3 * LB
ZERO_UNROLL = 8


def _sc_binning_kernel(pred_hbm, dur_hbm, ev_hbm, out_hbm,
                       pred_v, dur_v, ev_v, acc_v, red_v):
    wid = lax.axis_index("s") * NC + lax.axis_index("c")
    base = wid * PER_W
    pltpu.sync_copy(pred_hbm.at[pl.ds(base, PER_W)], pred_v)
    pltpu.sync_copy(dur_hbm.at[pl.ds(base, PER_W)], dur_v)
    pltpu.sync_copy(ev_hbm.at[pl.ds(base, PER_W)], ev_v)

    zeros16 = jnp.zeros((LANES,), jnp.float32)

    def zero_body(i, _):
        for k in range(ZERO_UNROLL):
            acc_v[pl.ds((i * ZERO_UNROLL + k) * LANES, LANES)] = zeros16
        return ()

    lax.fori_loop(0, ACC_WORDS // (LANES * ZERO_UNROLL), zero_body, ())

    lane_off = lax.broadcasted_iota(jnp.int32, (LANES,), 0) * NBUCKETS

    def body(i, _):
        s = pl.ds(i * LANES, LANES)
        y = pred_v[s]
        d = dur_v[s]
        e = ev_v[s].astype(jnp.float32)
        expy = jnp.exp(jnp.clip(y, -20.0, 20.0))
        addr = lane_off + d
        plsc.addupdate_scatter(acc_v, [addr], expy)
        plsc.addupdate_scatter(acc_v, [addr + LB], y * e)
        plsc.addupdate_scatter(acc_v, [addr + 2 * LB], e)
        return ()

    lax.fori_loop(0, CHUNKS, body, ())

    def red_body(c, _):
        col = c * LANES
        for q in range(3):
            tot = acc_v[pl.ds(q * LB + col, LANES)]
            for r in range(1, LANES):
                tot = tot + acc_v[pl.ds(q * LB + r * NBUCKETS + col, LANES)]
            red_v[q, pl.ds(col, LANES)] = tot
        return ()

    lax.fori_loop(0, NBUCKETS // LANES, red_body, ())
    pltpu.sync_copy(red_v, out_hbm.at[pl.ds(8 * wid, 8)])


def _sc_binning(pred, durations, events):
    mesh = plsc.VectorSubcoreMesh(core_axis_name="c", subcore_axis_name="s")
    kern = functools.partial(
        pl.kernel,
        mesh=mesh,
        out_type=jax.ShapeDtypeStruct((8 * NW, NBUCKETS), jnp.float32),
        scratch_types=[
            pltpu.VMEM((PER_W,), jnp.float32),
            pltpu.VMEM((PER_W,), jnp.int32),
            pltpu.VMEM((PER_W,), jnp.int32),
            pltpu.VMEM((ACC_WORDS,), jnp.float32),
            pltpu.VMEM((8, NBUCKETS), jnp.float32),
        ],
        compiler_params=pltpu.CompilerParams(needs_layout_passes=False),
    )(_sc_binning_kernel)
    return kern(pred, durations, events)


def _tc_finalize_kernel(x_ref, o_ref):
    x = x_ref[...]
    rows = x.shape[0]
    q = lax.broadcasted_iota(jnp.int32, (rows, 1), 0) % 8
    s_exp = jnp.sum(jnp.where(q == 0, x, 0.0), axis=0, keepdims=True)
    s_ye = jnp.sum(jnp.where(q == 1, x, 0.0), axis=0, keepdims=True)
    s_e = jnp.sum(jnp.where(q == 2, x, 0.0), axis=0, keepdims=True)
    jj = lax.broadcasted_iota(jnp.int32, (NBUCKETS, NBUCKETS), 0)
    ii = lax.broadcasted_iota(jnp.int32, (NBUCKETS, NBUCKETS), 1)
    tri = (jj >= ii).astype(jnp.float32)
    risk = jnp.dot(s_exp, tri, precision=lax.Precision.HIGHEST,
                   preferred_element_type=jnp.float32)
    ll = s_ye - s_e * jnp.log(jnp.maximum(risk, 1e-12))
    total_ll = jnp.sum(jnp.where(s_e > 0.0, ll, 0.0))
    n_events = jnp.maximum(jnp.sum(s_e), 1.0)
    o_ref[...] = jnp.broadcast_to(-total_ll / n_events, (1, 1))


def kernel(pred, durations, events):
    parts = _sc_binning(pred.reshape(-1).astype(jnp.float32),
                        durations.reshape(-1), events.reshape(-1))
    out = pl.pallas_call(
        _tc_finalize_kernel,
        out_shape=jax.ShapeDtypeStruct((1, 1), jnp.float32),
    )(parts)
    return out.reshape(1)


# speedup vs baseline: 4.6791x; 1.1231x over previous
import functools

import jax
import jax.numpy as jnp
from jax import lax
from jax.experimental import pallas as pl
from jax.experimental.pallas import tpu as pltpu
from jax.experimental.pallas import tpu_sc as plsc

N = 65536
NBUCKETS = 512
NC = 2
NS = 16
NW = NC * NS
PER_W = N // NW
LANES = 16
CHUNKS = PER_W // LANES
SLAB = 8


def _sc_binning_kernel(pred_hbm, dur_hbm, ev_hbm, out_hbm,
                       pred_v, dur_v, ev_v, acc_v):
    wid = lax.axis_index("s") * NC + lax.axis_index("c")
    base = wid * PER_W
    pltpu.sync_copy(pred_hbm.at[pl.ds(base, PER_W)], pred_v)
    pltpu.sync_copy(dur_hbm.at[pl.ds(base, PER_W)], dur_v)
    pltpu.sync_copy(ev_hbm.at[pl.ds(base, PER_W)], ev_v)

    zeros16 = jnp.zeros((LANES,), jnp.float32)

    def zero_body(i, _):
        for q in range(3):
            acc_v[q, pl.ds(i * LANES, LANES)] = zeros16
        return ()

    lax.fori_loop(0, NBUCKETS // LANES, zero_body, ())

    q0 = jnp.zeros((LANES,), jnp.int32)
    q1 = q0 + 1
    q2 = q0 + 2

    def body(i, _):
        s = pl.ds(i * LANES, LANES)
        y = pred_v[s]
        d = dur_v[s]
        e = ev_v[s].astype(jnp.float32)
        expy = jnp.exp(jnp.clip(y, -20.0, 20.0))
        plsc.addupdate_scatter(acc_v, [q0, d], expy)
        plsc.addupdate_scatter(acc_v, [q1, d], y * e)
        plsc.addupdate_scatter(acc_v, [q2, d], e)
        return ()

    lax.fori_loop(0, CHUNKS, body, ())
    pltpu.sync_copy(acc_v, out_hbm.at[pl.ds(SLAB * wid, SLAB)])


def _sc_binning(pred, durations, events):
    mesh = plsc.VectorSubcoreMesh(core_axis_name="c", subcore_axis_name="s")
    kern = functools.partial(
        pl.kernel,
        mesh=mesh,
        out_type=jax.ShapeDtypeStruct((SLAB * NW, NBUCKETS), jnp.float32),
        scratch_types=[
            pltpu.VMEM((PER_W,), jnp.float32),
            pltpu.VMEM((PER_W,), jnp.int32),
            pltpu.VMEM((PER_W,), jnp.int32),
            pltpu.VMEM((SLAB, NBUCKETS), jnp.float32),
        ],
        compiler_params=pltpu.CompilerParams(needs_layout_passes=False),
    )(_sc_binning_kernel)
    return kern(pred, durations, events)


def _tc_finalize_kernel(x_ref, o_ref):
    x = x_ref[...]
    rows = x.shape[0]
    q = lax.broadcasted_iota(jnp.int32, (rows, 1), 0) % SLAB
    s_exp = jnp.sum(jnp.where(q == 0, x, 0.0), axis=0, keepdims=True)
    s_ye = jnp.sum(jnp.where(q == 1, x, 0.0), axis=0, keepdims=True)
    s_e = jnp.sum(jnp.where(q == 2, x, 0.0), axis=0, keepdims=True)
    jj = lax.broadcasted_iota(jnp.int32, (NBUCKETS, NBUCKETS), 0)
    ii = lax.broadcasted_iota(jnp.int32, (NBUCKETS, NBUCKETS), 1)
    tri = (jj >= ii).astype(jnp.float32)
    risk = jnp.dot(s_exp, tri, precision=lax.Precision.HIGHEST,
                   preferred_element_type=jnp.float32)
    ll = s_ye - s_e * jnp.log(jnp.maximum(risk, 1e-12))
    total_ll = jnp.sum(jnp.where(s_e > 0.0, ll, 0.0))
    n_events = jnp.maximum(jnp.sum(s_e), 1.0)
    o_ref[...] = jnp.broadcast_to(-total_ll / n_events, (1, 1))


def kernel(pred, durations, events):
    parts = _sc_binning(pred.reshape(-1).astype(jnp.float32),
                        durations.reshape(-1), events.reshape(-1))
    out = pl.pallas_call(
        _tc_finalize_kernel,
        out_shape=jax.ShapeDtypeStruct((1, 1), jnp.float32),
    )(parts)
    return out.reshape(1)


# speedup vs baseline: 5.1318x; 1.0967x over previous
import functools

import jax
import jax.numpy as jnp
from jax import lax
from jax.experimental import pallas as pl
from jax.experimental.pallas import tpu as pltpu
from jax.experimental.pallas import tpu_sc as plsc

N = 65536
NBUCKETS = 512
NC = 2
NS = 16
NW = NC * NS
PER_W = N // NW
LANES = 16
CHUNKS = PER_W // LANES
SLAB = 8


def _sc_binning_kernel(pred_hbm, dur_hbm, ev_hbm, out_hbm,
                       pred_v, dur_v, ev_v, acc_v, sem):
    wid = lax.axis_index("s") * NC + lax.axis_index("c")
    base = wid * PER_W
    cp_p = pltpu.async_copy(pred_hbm.at[pl.ds(base, PER_W)], pred_v, sem)
    cp_d = pltpu.async_copy(dur_hbm.at[pl.ds(base, PER_W)], dur_v, sem)
    cp_e = pltpu.async_copy(ev_hbm.at[pl.ds(base, PER_W)], ev_v, sem)

    zeros16 = jnp.zeros((LANES,), jnp.float32)

    @plsc.parallel_loop(0, NBUCKETS // LANES, unroll=4)
    def zero_body(i):
        for q in range(3):
            acc_v[q, pl.ds(i * LANES, LANES)] = zeros16

    cp_p.wait()
    cp_d.wait()
    cp_e.wait()

    q0 = jnp.zeros((LANES,), jnp.int32)
    q1 = q0 + 1
    q2 = q0 + 2

    @plsc.parallel_loop(0, CHUNKS, unroll=4)
    def body(i):
        s = pl.ds(i * LANES, LANES)
        y = pred_v[s]
        d = dur_v[s]
        e = ev_v[s].astype(jnp.float32)
        expy = jnp.exp(jnp.clip(y, -20.0, 20.0))
        plsc.addupdate_scatter(acc_v, [q0, d], expy)
        plsc.addupdate_scatter(acc_v, [q1, d], y * e)
        plsc.addupdate_scatter(acc_v, [q2, d], e)

    pltpu.sync_copy(acc_v, out_hbm.at[pl.ds(SLAB * wid, SLAB)])


def _sc_binning(pred, durations, events):
    mesh = plsc.VectorSubcoreMesh(core_axis_name="c", subcore_axis_name="s")
    kern = functools.partial(
        pl.kernel,
        mesh=mesh,
        out_type=jax.ShapeDtypeStruct((SLAB * NW, NBUCKETS), jnp.float32),
        scratch_types=[
            pltpu.VMEM((PER_W,), jnp.float32),
            pltpu.VMEM((PER_W,), jnp.int32),
            pltpu.VMEM((PER_W,), jnp.int32),
            pltpu.VMEM((SLAB, NBUCKETS), jnp.float32),
            pltpu.SemaphoreType.DMA,
        ],
        compiler_params=pltpu.CompilerParams(needs_layout_passes=False),
    )(_sc_binning_kernel)
    return kern(pred, durations, events)


def _tc_finalize_kernel(x_ref, o_ref):
    x = x_ref[...]
    rows = x.shape[0]
    q = lax.broadcasted_iota(jnp.int32, (rows, 1), 0) % SLAB
    s_exp = jnp.sum(jnp.where(q == 0, x, 0.0), axis=0, keepdims=True)
    s_ye = jnp.sum(jnp.where(q == 1, x, 0.0), axis=0, keepdims=True)
    s_e = jnp.sum(jnp.where(q == 2, x, 0.0), axis=0, keepdims=True)
    jj = lax.broadcasted_iota(jnp.int32, (NBUCKETS, NBUCKETS), 0)
    ii = lax.broadcasted_iota(jnp.int32, (NBUCKETS, NBUCKETS), 1)
    tri = (jj >= ii).astype(jnp.float32)
    risk = jnp.dot(s_exp, tri, precision=lax.Precision.HIGHEST,
                   preferred_element_type=jnp.float32)
    ll = s_ye - s_e * jnp.log(jnp.maximum(risk, 1e-12))
    total_ll = jnp.sum(jnp.where(s_e > 0.0, ll, 0.0))
    n_events = jnp.maximum(jnp.sum(s_e), 1.0)
    o_ref[...] = jnp.broadcast_to(-total_ll / n_events, (1, 1))


def kernel(pred, durations, events):
    parts = _sc_binning(pred.reshape(-1).astype(jnp.float32),
                        durations.reshape(-1), events.reshape(-1))
    out = pl.pallas_call(
        _tc_finalize_kernel,
        out_shape=jax.ShapeDtypeStruct((1, 1), jnp.float32),
    )(parts)
    return out.reshape(1)


# speedup vs baseline: 5.1338x; 1.0004x over previous
import functools

import jax
import jax.numpy as jnp
from jax import lax
from jax.experimental import pallas as pl
from jax.experimental.pallas import tpu as pltpu
from jax.experimental.pallas import tpu_sc as plsc

N = 65536
NBUCKETS = 512
NC = 2
NS = 16
NW = NC * NS
PER_W = N // NW
LANES = 16
CHUNKS = PER_W // LANES
SLAB = 8


def _sc_binning_kernel(pred_hbm, dur_hbm, ev_hbm, out_hbm,
                       pred_v, dur_v, ev_v, acc_v, sem):
    wid = lax.axis_index("s") * NC + lax.axis_index("c")
    base = wid * PER_W
    cp_p = pltpu.async_copy(pred_hbm.at[pl.ds(base, PER_W)], pred_v, sem)
    cp_d = pltpu.async_copy(dur_hbm.at[pl.ds(base, PER_W)], dur_v, sem)
    cp_e = pltpu.async_copy(ev_hbm.at[pl.ds(base, PER_W)], ev_v, sem)

    zeros16 = jnp.zeros((LANES,), jnp.float32)

    @plsc.parallel_loop(0, NBUCKETS // LANES, unroll=4)
    def zero_body(i):
        for q in range(3):
            acc_v[q, pl.ds(i * LANES, LANES)] = zeros16

    cp_p.wait()
    cp_d.wait()
    cp_e.wait()

    q0 = jnp.zeros((LANES,), jnp.int32)
    q1 = q0 + 1
    q2 = q0 + 2

    @plsc.parallel_loop(0, CHUNKS, unroll=8)
    def body(i):
        s = pl.ds(i * LANES, LANES)
        y = pred_v[s]
        d = dur_v[s]
        e = ev_v[s].astype(jnp.float32)
        expy = jnp.exp(jnp.clip(y, -20.0, 20.0))
        plsc.addupdate_scatter(acc_v, [q0, d], expy)
        plsc.addupdate_scatter(acc_v, [q1, d], y * e)
        plsc.addupdate_scatter(acc_v, [q2, d], e)

    pltpu.sync_copy(acc_v, out_hbm.at[pl.ds(SLAB * wid, SLAB)])


def _sc_binning(pred, durations, events):
    mesh = plsc.VectorSubcoreMesh(core_axis_name="c", subcore_axis_name="s")
    kern = functools.partial(
        pl.kernel,
        mesh=mesh,
        out_type=jax.ShapeDtypeStruct((SLAB * NW, NBUCKETS), jnp.float32),
        scratch_types=[
            pltpu.VMEM((PER_W,), jnp.float32),
            pltpu.VMEM((PER_W,), jnp.int32),
            pltpu.VMEM((PER_W,), jnp.int32),
            pltpu.VMEM((SLAB, NBUCKETS), jnp.float32),
            pltpu.SemaphoreType.DMA,
        ],
        compiler_params=pltpu.CompilerParams(needs_layout_passes=False),
    )(_sc_binning_kernel)
    return kern(pred, durations, events)


def _tc_finalize_kernel(x_ref, o_ref):
    x = x_ref[...]
    rows = x.shape[0]
    q = lax.broadcasted_iota(jnp.int32, (rows, 1), 0) % SLAB
    s_exp = jnp.sum(jnp.where(q == 0, x, 0.0), axis=0, keepdims=True)
    s_ye = jnp.sum(jnp.where(q == 1, x, 0.0), axis=0, keepdims=True)
    s_e = jnp.sum(jnp.where(q == 2, x, 0.0), axis=0, keepdims=True)
    col = lax.broadcasted_iota(jnp.int32, (1, NBUCKETS), 1)
    risk = s_exp
    k = 1
    while k < NBUCKETS:
        shifted = pltpu.roll(risk, NBUCKETS - k, 1)
        risk = risk + jnp.where(col < NBUCKETS - k, shifted, 0.0)
        k *= 2
    ll = s_ye - s_e * jnp.log(jnp.maximum(risk, 1e-12))
    total_ll = jnp.sum(jnp.where(s_e > 0.0, ll, 0.0))
    n_events = jnp.maximum(jnp.sum(s_e), 1.0)
    o_ref[...] = jnp.broadcast_to(-total_ll / n_events, (1, 1))


def kernel(pred, durations, events):
    parts = _sc_binning(pred.reshape(-1).astype(jnp.float32),
                        durations.reshape(-1), events.reshape(-1))
    out = pl.pallas_call(
        _tc_finalize_kernel,
        out_shape=jax.ShapeDtypeStruct((1, 1), jnp.float32),
    )(parts)
    return out.reshape(1)
